# Initial kernel scaffold; baseline (speedup 1.0000x reference)
import functools

import jax
import jax.numpy as jnp
from jax import lax
from jax.experimental import pallas as pl
from jax.experimental.pallas import tpu as pltpu
from jax.experimental.pallas import tpu_sc as plsc

NC = 2
NS = 16
NW = NC * NS

BATCH = 4096
HIST = 200
EMBED = 32
VOCAB = 1000000

ROWS_PER_W = BATCH // NW
CHUNK = 100
CHUNKS_PER_W = ROWS_PER_W * 2
NBUF = 4


def _body(idx_hbm, table_hbm, out_hbm, idx_v, bufs, out_v, s0, s1, s2, s3):
    sems = (s0, s1, s2, s3)
    wid = lax.axis_index("s") * NC + lax.axis_index("c")

    pltpu.sync_copy(idx_hbm.at[wid], idx_v)

    for i in range(NBUF):
        pltpu.async_copy(table_hbm.at[idx_v.at[i]], bufs.at[i], sems[i])

    def sum_chunk(i, acc0, acc1):
        for r in range(CHUNK):
            acc0 = acc0 + bufs[i, r, 0:16]
            acc1 = acc1 + bufs[i, r, 16:32]
        return acc0, acc1

    zero = jnp.zeros((16,), jnp.float32)

    def j_body(j, carry):
        for half in range(2):
            row = 2 * j + half
            acc0, acc1 = zero, zero
            for i in (2 * half, 2 * half + 1):
                pltpu.make_async_copy(
                    table_hbm.at[idx_v.at[i]], bufs.at[i], sems[i]
                ).wait()
                acc0, acc1 = sum_chunk(i, acc0, acc1)

                @pl.when(j < CHUNKS_PER_W // NBUF - 1)
                def _():
                    nxt = NBUF * j + NBUF + i
                    pltpu.async_copy(
                        table_hbm.at[idx_v.at[nxt]], bufs.at[i], sems[i]
                    )

            out_v[row, 0:16] = acc0
            out_v[row, 16:32] = acc1
        return carry

    lax.fori_loop(0, CHUNKS_PER_W // NBUF, j_body, 0)

    pltpu.sync_copy(out_v, out_hbm.at[wid])


@jax.jit
def _embed_sum(idx, table):
    mesh = plsc.VectorSubcoreMesh(
        core_axis_name="c", subcore_axis_name="s", num_cores=NC, num_subcores=NS
    )
    f = functools.partial(
        pl.kernel,
        mesh=mesh,
        out_type=jax.ShapeDtypeStruct((NW, ROWS_PER_W, EMBED), jnp.float32),
        scratch_types=[
            pltpu.VMEM((CHUNKS_PER_W, CHUNK), jnp.int32),
            pltpu.VMEM((NBUF, CHUNK, EMBED), jnp.float32),
            pltpu.VMEM((ROWS_PER_W, EMBED), jnp.float32),
            pltpu.SemaphoreType.DMA,
            pltpu.SemaphoreType.DMA,
            pltpu.SemaphoreType.DMA,
            pltpu.SemaphoreType.DMA,
        ],
    )(_body)
    return f(idx, table)


def kernel(ancestor_idx, embed_weight):
    idx = ancestor_idx.astype(jnp.int32).reshape(NW, CHUNKS_PER_W, CHUNK)
    out = _embed_sum(idx, embed_weight)
    return out.reshape(BATCH, EMBED)


# speedup vs baseline: 2.3625x; 2.3625x over previous
import functools

import jax
import jax.numpy as jnp
from jax import lax
from jax.experimental import pallas as pl
from jax.experimental.pallas import tpu as pltpu
from jax.experimental.pallas import tpu_sc as plsc

NC = 2
NS = 16
NW = NC * NS

BATCH = 4096
HIST = 200
EMBED = 32
VOCAB = 1000000

ROWS_PER_W = BATCH // NW
CHUNK = 100
CHUNKS_PER_W = ROWS_PER_W * 2
NBUF = 4


def _body(idx_hbm, table_hbm, out_hbm, idx_v, bufs, out_v, s0, s1, s2, s3):
    sems = (s0, s1, s2, s3)
    wid = lax.axis_index("s") * NC + lax.axis_index("c")

    pltpu.sync_copy(idx_hbm.at[wid], idx_v)

    for i in range(NBUF):
        pltpu.async_copy(table_hbm.at[idx_v.at[i]], bufs.at[i], sems[i])

    def sum_chunk(i, acc0, acc1):
        for r in range(CHUNK):
            acc0 = acc0 + bufs[i, r, 0:16]
            acc1 = acc1 + bufs[i, r, 16:32]
        return acc0, acc1

    zero = jnp.zeros((16,), jnp.float32)

    def j_body(j, carry):
        for half in range(2):
            row = 2 * j + half
            acc0, acc1 = zero, zero
            for i in (2 * half, 2 * half + 1):
                pltpu.make_async_copy(
                    table_hbm.at[idx_v.at[i]], bufs.at[i], sems[i]
                ).wait()
                acc0, acc1 = sum_chunk(i, acc0, acc1)

                @pl.when(j < CHUNKS_PER_W // NBUF - 1)
                def _():
                    nxt = NBUF * j + NBUF + i
                    pltpu.async_copy(
                        table_hbm.at[idx_v.at[nxt]], bufs.at[i], sems[i]
                    )

            out_v[row, 0:16] = acc0
            out_v[row, 16:32] = acc1
        return carry

    lax.fori_loop(0, CHUNKS_PER_W // NBUF, j_body, 0)

    pltpu.sync_copy(out_v, out_hbm.at[wid])


@jax.jit
def _embed_sum(idx, table):
    mesh = plsc.VectorSubcoreMesh(
        core_axis_name="c", subcore_axis_name="s", num_cores=NC, num_subcores=NS
    )
    f = functools.partial(
        pl.kernel,
        mesh=mesh,
        out_type=jax.ShapeDtypeStruct((NW, ROWS_PER_W, EMBED), jnp.float32),
        scratch_types=[
            pltpu.VMEM((CHUNKS_PER_W, CHUNK), jnp.int32),
            pltpu.VMEM((NBUF, CHUNK, EMBED), jnp.float32),
            pltpu.VMEM((ROWS_PER_W, EMBED), jnp.float32),
            pltpu.SemaphoreType.DMA,
            pltpu.SemaphoreType.DMA,
            pltpu.SemaphoreType.DMA,
            pltpu.SemaphoreType.DMA,
        ],
        compiler_params=pltpu.CompilerParams(use_tc_tiling_on_sc=False),
    )(_body)
    return f(idx, table)


def kernel(ancestor_idx, embed_weight):
    idx = ancestor_idx.astype(jnp.int32).reshape(NW, CHUNKS_PER_W, CHUNK)
    out = _embed_sum(idx, embed_weight)
    return out.reshape(BATCH, EMBED)
